# baseline (speedup 1.0000x reference)
import functools

import jax
import jax.numpy as jnp
from jax import lax
from jax.experimental import pallas as pl
from jax.experimental.pallas import tpu as pltpu
from jax.experimental.pallas import tpu_sc as plsc

N_NODES = 10000
N_EDGES = 320000
D_FEAT = 128
DW = D_FEAT // 2

NW = 32
C = 128
NCHUNKS = N_EDGES // C
BASE_TRIPS = NCHUNKS // NW
EXTRA = NCHUNKS % NW
NBUF = 3
NSLOT = 81

_mesh = plsc.VectorSubcoreMesh(core_axis_name="c", subcore_axis_name="s")


def _prep_body(x_ref, xb_ref):
    xb_ref[...] = x_ref[...].astype(jnp.bfloat16)


def _prep(x_perm):
    return pl.pallas_call(
        _prep_body,
        out_shape=jax.ShapeDtypeStruct((N_NODES, D_FEAT), jnp.bfloat16),
    )(x_perm)


_scratch = []
for _b in range(NBUF):
    _scratch += [
        pltpu.VMEM((C,), jnp.int32),
        pltpu.VMEM((C,), jnp.int32),
        pltpu.VMEM((C, DW), jnp.int32),
        pltpu.VMEM((C, DW), jnp.int32),
        pltpu.VMEM((C, D_FEAT), jnp.float32),
        pltpu.SemaphoreType.DMA,
        pltpu.SemaphoreType.DMA,
        pltpu.SemaphoreType.DMA,
    ]

_HI = -65536


@functools.partial(
    pl.kernel,
    out_type=jax.ShapeDtypeStruct((N_EDGES, D_FEAT), jnp.float32),
    mesh=_mesh,
    scratch_types=_scratch,
    compiler_params=pltpu.CompilerParams(use_tc_tiling_on_sc=False),
)
def _edge_grad(xw_hbm, src_hbm, dst_hbm, out_hbm, *scr):
    idx_s = [scr[8 * b + 0] for b in range(NBUF)]
    idx_d = [scr[8 * b + 1] for b in range(NBUF)]
    rows_s = [scr[8 * b + 2] for b in range(NBUF)]
    rows_d = [scr[8 * b + 3] for b in range(NBUF)]
    rows_f = [scr[8 * b + 4] for b in range(NBUF)]
    sem_i = [scr[8 * b + 5] for b in range(NBUF)]
    sem_g = [scr[8 * b + 6] for b in range(NBUF)]
    sem_o = [scr[8 * b + 7] for b in range(NBUF)]

    wid = lax.axis_index("s") * 2 + lax.axis_index("c")
    chunk0 = BASE_TRIPS * wid + lax.min(wid, EXTRA)
    trips = BASE_TRIPS + jnp.where(wid < EXTRA, 1, 0)

    def group_body(g, carry):
        js = [g * NBUF + b for b in range(NBUF)]
        live = [js[b] < trips for b in range(NBUF)]
        bases = [(chunk0 + js[b]) * C for b in range(NBUF)]
        for b in range(NBUF):
            @pl.when(live[b])
            def _(b=b):
                pltpu.async_copy(src_hbm.at[pl.ds(bases[b], C)], idx_s[b], sem_i[b])
                pltpu.async_copy(dst_hbm.at[pl.ds(bases[b], C)], idx_d[b], sem_i[b])
        for b in range(NBUF):
            @pl.when(live[b])
            def _(b=b):
                pltpu.make_async_copy(
                    src_hbm.at[pl.ds(bases[b], C)], idx_s[b], sem_i[b]).wait()
                pltpu.make_async_copy(
                    dst_hbm.at[pl.ds(bases[b], C)], idx_d[b], sem_i[b]).wait()
                pltpu.async_copy(xw_hbm.at[idx_s[b]], rows_s[b], sem_g[b])
                pltpu.async_copy(xw_hbm.at[idx_d[b]], rows_d[b], sem_g[b])
        for b in range(NBUF):
            @pl.when(live[b])
            def _(b=b):
                pltpu.make_async_copy(
                    xw_hbm.at[idx_s[b]], rows_s[b], sem_g[b]).wait()
                pltpu.make_async_copy(
                    xw_hbm.at[idx_d[b]], rows_d[b], sem_g[b]).wait()

                def row_body(r, rcarry):
                    for t in range(DW // 16):
                        ws = rows_s[b][r, pl.ds(16 * t, 16)]
                        wd = rows_d[b][r, pl.ds(16 * t, 16)]
                        lo = (lax.bitcast_convert_type(wd << 16, jnp.float32)
                              - lax.bitcast_convert_type(ws << 16, jnp.float32))
                        hi = (lax.bitcast_convert_type(wd & _HI, jnp.float32)
                              - lax.bitcast_convert_type(ws & _HI, jnp.float32))
                        rows_f[b][r, pl.ds(32 * t, 16)] = lo
                        rows_f[b][r, pl.ds(32 * t + 16, 16)] = hi
                    return rcarry

                lax.fori_loop(0, C, row_body, 0, unroll=2)
                pltpu.async_copy(rows_f[b], out_hbm.at[pl.ds(bases[b], C)], sem_o[b])
        for b in range(NBUF):
            @pl.when(live[b])
            def _(b=b):
                pltpu.make_async_copy(
                    rows_f[b], out_hbm.at[pl.ds(bases[b], C)], sem_o[b]).wait()
        return carry

    lax.fori_loop(0, NSLOT // NBUF, group_body, 0, unroll=False)


def kernel(x, edge_index):
    x_perm = x.reshape(N_NODES, D_FEAT // 32, 2, 16).swapaxes(2, 3)
    x_perm = x_perm.reshape(N_NODES, D_FEAT)
    xb = _prep(x_perm)
    xw = jax.lax.bitcast_convert_type(xb.reshape(N_NODES, DW, 2), jnp.int32)
    src = edge_index[0]
    dst = edge_index[1]
    return _edge_grad(xw, src, dst)


# speedup vs baseline: 1.8430x; 1.8430x over previous
import functools

import jax
import jax.numpy as jnp
from jax import lax
from jax.experimental import pallas as pl
from jax.experimental.pallas import tpu as pltpu
from jax.experimental.pallas import tpu_sc as plsc

N_NODES = 10000
N_EDGES = 320000
D_FEAT = 128
DW = D_FEAT // 2

NW = 32
C = 128
NCHUNKS = N_EDGES // C
BASE_TRIPS = NCHUNKS // NW
EXTRA = NCHUNKS % NW
NBUF = 3
NSLOT = 81

_mesh = plsc.VectorSubcoreMesh(core_axis_name="c", subcore_axis_name="s")


def _prep_body(x_ref, xb_ref):
    xb_ref[...] = x_ref[...].astype(jnp.bfloat16)


def _prep(x_perm):
    return pl.pallas_call(
        _prep_body,
        out_shape=jax.ShapeDtypeStruct((N_NODES, D_FEAT), jnp.bfloat16),
    )(x_perm)


_scratch = []
for _b in range(NBUF):
    _scratch += [
        pltpu.VMEM((C,), jnp.int32),
        pltpu.VMEM((C,), jnp.int32),
        pltpu.VMEM((C, DW), jnp.int32),
        pltpu.VMEM((C, DW), jnp.int32),
        pltpu.VMEM((C, D_FEAT), jnp.float32),
        pltpu.SemaphoreType.DMA,
        pltpu.SemaphoreType.DMA,
        pltpu.SemaphoreType.DMA,
    ]

_HI = -65536


@functools.partial(
    pl.kernel,
    out_type=jax.ShapeDtypeStruct((N_EDGES, D_FEAT), jnp.float32),
    mesh=_mesh,
    scratch_types=_scratch,
    compiler_params=pltpu.CompilerParams(use_tc_tiling_on_sc=False),
)
def _edge_grad(xw_hbm, src_hbm, dst_hbm, out_hbm, *scr):
    idx_s = [scr[8 * b + 0] for b in range(NBUF)]
    idx_d = [scr[8 * b + 1] for b in range(NBUF)]
    rows_s = [scr[8 * b + 2] for b in range(NBUF)]
    rows_d = [scr[8 * b + 3] for b in range(NBUF)]
    rows_f = [scr[8 * b + 4] for b in range(NBUF)]
    sem_i = [scr[8 * b + 5] for b in range(NBUF)]
    sem_g = [scr[8 * b + 6] for b in range(NBUF)]
    sem_o = [scr[8 * b + 7] for b in range(NBUF)]

    wid = lax.axis_index("s") * 2 + lax.axis_index("c")
    chunk0 = BASE_TRIPS * wid + lax.min(wid, EXTRA)
    trips = BASE_TRIPS + jnp.where(wid < EXTRA, 1, 0)

    def group_body(g, carry):
        js = [g * NBUF + b for b in range(NBUF)]
        live = [js[b] < trips for b in range(NBUF)]
        bases = [(chunk0 + js[b]) * C for b in range(NBUF)]
        for b in range(NBUF):
            @pl.when(live[b])
            def _(b=b):
                pltpu.async_copy(src_hbm.at[pl.ds(bases[b], C)], idx_s[b], sem_i[b])
                pltpu.async_copy(dst_hbm.at[pl.ds(bases[b], C)], idx_d[b], sem_i[b])
        for b in range(NBUF):
            @pl.when(live[b])
            def _(b=b):
                pltpu.make_async_copy(
                    src_hbm.at[pl.ds(bases[b], C)], idx_s[b], sem_i[b]).wait()
                pltpu.make_async_copy(
                    dst_hbm.at[pl.ds(bases[b], C)], idx_d[b], sem_i[b]).wait()
                pltpu.async_copy(xw_hbm.at[idx_s[b]], rows_s[b], sem_g[b])
                pltpu.async_copy(xw_hbm.at[idx_d[b]], rows_d[b], sem_g[b])
        for b in range(NBUF):
            @pl.when(live[b])
            def _(b=b):
                pltpu.make_async_copy(
                    xw_hbm.at[idx_s[b]], rows_s[b], sem_g[b]).wait()
                pltpu.make_async_copy(
                    xw_hbm.at[idx_d[b]], rows_d[b], sem_g[b]).wait()

                @plsc.parallel_loop(0, C, unroll=4)
                def row_body(r):
                    for t in range(DW // 16):
                        ws = rows_s[b][r, pl.ds(16 * t, 16)]
                        wd = rows_d[b][r, pl.ds(16 * t, 16)]
                        lo = (lax.bitcast_convert_type(wd << 16, jnp.float32)
                              - lax.bitcast_convert_type(ws << 16, jnp.float32))
                        hi = (lax.bitcast_convert_type(wd & _HI, jnp.float32)
                              - lax.bitcast_convert_type(ws & _HI, jnp.float32))
                        rows_f[b][r, pl.ds(32 * t, 16)] = lo
                        rows_f[b][r, pl.ds(32 * t + 16, 16)] = hi
                pltpu.async_copy(rows_f[b], out_hbm.at[pl.ds(bases[b], C)], sem_o[b])
        for b in range(NBUF):
            @pl.when(live[b])
            def _(b=b):
                pltpu.make_async_copy(
                    rows_f[b], out_hbm.at[pl.ds(bases[b], C)], sem_o[b]).wait()
        return carry

    lax.fori_loop(0, NSLOT // NBUF, group_body, 0, unroll=False)


def kernel(x, edge_index):
    x_perm = x.reshape(N_NODES, D_FEAT // 32, 2, 16).swapaxes(2, 3)
    x_perm = x_perm.reshape(N_NODES, D_FEAT)
    xb = _prep(x_perm)
    xw = jax.lax.bitcast_convert_type(xb.reshape(N_NODES, DW, 2), jnp.int32)
    src = edge_index[0]
    dst = edge_index[1]
    return _edge_grad(xw, src, dst)


# speedup vs baseline: 2.0601x; 1.1178x over previous
import functools

import jax
import jax.numpy as jnp
from jax import lax
from jax.experimental import pallas as pl
from jax.experimental.pallas import tpu as pltpu
from jax.experimental.pallas import tpu_sc as plsc

N_NODES = 10000
N_EDGES = 320000
D_FEAT = 128

NW = 32
C = 128
NCHUNKS = N_EDGES // C
BASE_TRIPS = NCHUNKS // NW
EXTRA = NCHUNKS % NW
NBUF = 7
NSLOT = 84

_mesh = plsc.VectorSubcoreMesh(core_axis_name="c", subcore_axis_name="s")


def _neg_body(x_ref, o_ref):
    o_ref[...] = -x_ref[...]


def _negate(x):
    return pl.pallas_call(
        _neg_body,
        out_shape=jax.ShapeDtypeStruct((N_NODES, D_FEAT), jnp.float32),
    )(x)


_scratch = []
for _b in range(NBUF):
    _scratch += [
        pltpu.VMEM((C,), jnp.int32),
        pltpu.VMEM((C,), jnp.int32),
        pltpu.VMEM((C, D_FEAT), jnp.float32),
        pltpu.SemaphoreType.DMA,
        pltpu.SemaphoreType.DMA,
        pltpu.SemaphoreType.DMA,
        pltpu.SemaphoreType.DMA,
    ]


@functools.partial(
    pl.kernel,
    out_type=jax.ShapeDtypeStruct((N_EDGES, D_FEAT), jnp.float32),
    mesh=_mesh,
    scratch_types=_scratch,
)
def _edge_grad(x_hbm, negx_hbm, src_hbm, dst_hbm, out_hbm, *scr):
    idx_s = [scr[7 * b + 0] for b in range(NBUF)]
    idx_d = [scr[7 * b + 1] for b in range(NBUF)]
    rows = [scr[7 * b + 2] for b in range(NBUF)]
    sem_i = [scr[7 * b + 3] for b in range(NBUF)]
    sem_g = [scr[7 * b + 4] for b in range(NBUF)]
    sem_a = [scr[7 * b + 5] for b in range(NBUF)]
    sem_o = [scr[7 * b + 6] for b in range(NBUF)]

    wid = lax.axis_index("s") * 2 + lax.axis_index("c")
    chunk0 = BASE_TRIPS * wid + lax.min(wid, EXTRA)
    trips = BASE_TRIPS + jnp.where(wid < EXTRA, 1, 0)

    def group_body(g, carry):
        js = [g * NBUF + b for b in range(NBUF)]
        live = [js[b] < trips for b in range(NBUF)]
        bases = [(chunk0 + js[b]) * C for b in range(NBUF)]
        for b in range(NBUF):
            @pl.when(live[b])
            def _(b=b):
                pltpu.async_copy(src_hbm.at[pl.ds(bases[b], C)], idx_s[b], sem_i[b])
                pltpu.async_copy(dst_hbm.at[pl.ds(bases[b], C)], idx_d[b], sem_i[b])
        for b in range(NBUF):
            @pl.when(live[b])
            def _(b=b):
                pltpu.make_async_copy(
                    src_hbm.at[pl.ds(bases[b], C)], idx_s[b], sem_i[b]).wait()
                pltpu.make_async_copy(
                    dst_hbm.at[pl.ds(bases[b], C)], idx_d[b], sem_i[b]).wait()
                pltpu.async_copy(x_hbm.at[idx_d[b]], rows[b], sem_g[b])
        for b in range(NBUF):
            @pl.when(live[b])
            def _(b=b):
                pltpu.make_async_copy(x_hbm.at[idx_d[b]], rows[b], sem_g[b]).wait()
                pltpu.make_async_copy(
                    negx_hbm.at[idx_s[b]], rows[b], sem_a[b]).start(add=True)
        for b in range(NBUF):
            @pl.when(live[b])
            def _(b=b):
                pltpu.make_async_copy(negx_hbm.at[idx_s[b]], rows[b], sem_a[b]).wait()
                pltpu.async_copy(rows[b], out_hbm.at[pl.ds(bases[b], C)], sem_o[b])
        for b in range(NBUF):
            @pl.when(live[b])
            def _(b=b):
                pltpu.make_async_copy(
                    rows[b], out_hbm.at[pl.ds(bases[b], C)], sem_o[b]).wait()
        return carry

    lax.fori_loop(0, NSLOT // NBUF, group_body, 0, unroll=False)


def kernel(x, edge_index):
    negx = _negate(x)
    src = edge_index[0]
    dst = edge_index[1]
    return _edge_grad(x, negx, src, dst)


# speedup vs baseline: 2.0668x; 1.0033x over previous
import functools

import jax
import jax.numpy as jnp
from jax import lax
from jax.experimental import pallas as pl
from jax.experimental.pallas import tpu as pltpu
from jax.experimental.pallas import tpu_sc as plsc

N_NODES = 10000
N_EDGES = 320000
D_FEAT = 128

NW = 32
C = 128
NCHUNKS = N_EDGES // C
BASE_TRIPS = NCHUNKS // NW
EXTRA = NCHUNKS % NW
NBUF = 6
NSLOT = 84

_mesh = plsc.VectorSubcoreMesh(core_axis_name="c", subcore_axis_name="s")


def _neg_body(x_ref, o_ref):
    o_ref[...] = -x_ref[...]


def _negate(x):
    return pl.pallas_call(
        _neg_body,
        out_shape=jax.ShapeDtypeStruct((N_NODES, D_FEAT), jnp.float32),
    )(x)


_scratch = []
for _b in range(NBUF):
    _scratch += [
        pltpu.VMEM((C,), jnp.int32),
        pltpu.VMEM((C,), jnp.int32),
        pltpu.VMEM((C, D_FEAT), jnp.float32),
        pltpu.SemaphoreType.DMA,
        pltpu.SemaphoreType.DMA,
        pltpu.SemaphoreType.DMA,
        pltpu.SemaphoreType.DMA,
    ]


@functools.partial(
    pl.kernel,
    out_type=jax.ShapeDtypeStruct((N_EDGES, D_FEAT), jnp.float32),
    mesh=_mesh,
    scratch_types=_scratch,
)
def _edge_grad(x_hbm, negx_hbm, src_hbm, dst_hbm, out_hbm, *scr):
    idx_s = [scr[7 * b + 0] for b in range(NBUF)]
    idx_d = [scr[7 * b + 1] for b in range(NBUF)]
    rows = [scr[7 * b + 2] for b in range(NBUF)]
    sem_i = [scr[7 * b + 3] for b in range(NBUF)]
    sem_g = [scr[7 * b + 4] for b in range(NBUF)]
    sem_a = [scr[7 * b + 5] for b in range(NBUF)]
    sem_o = [scr[7 * b + 6] for b in range(NBUF)]

    wid = lax.axis_index("s") * 2 + lax.axis_index("c")
    chunk0 = BASE_TRIPS * wid + lax.min(wid, EXTRA)
    trips = BASE_TRIPS + jnp.where(wid < EXTRA, 1, 0)

    def group_body(g, carry):
        js = [g * NBUF + b for b in range(NBUF)]
        live = [js[b] < trips for b in range(NBUF)]
        bases = [(chunk0 + js[b]) * C for b in range(NBUF)]
        for b in range(NBUF):
            @pl.when((js[b] >= NBUF) & live[b])
            def _(b=b):
                pltpu.make_async_copy(
                    rows[b], out_hbm.at[pl.ds(bases[b] - NBUF * C, C)],
                    sem_o[b]).wait()
        for b in range(NBUF):
            @pl.when(live[b])
            def _(b=b):
                pltpu.async_copy(src_hbm.at[pl.ds(bases[b], C)], idx_s[b], sem_i[b])
                pltpu.async_copy(dst_hbm.at[pl.ds(bases[b], C)], idx_d[b], sem_i[b])
        for b in range(NBUF):
            @pl.when(live[b])
            def _(b=b):
                pltpu.make_async_copy(
                    src_hbm.at[pl.ds(bases[b], C)], idx_s[b], sem_i[b]).wait()
                pltpu.make_async_copy(
                    dst_hbm.at[pl.ds(bases[b], C)], idx_d[b], sem_i[b]).wait()
                pltpu.async_copy(x_hbm.at[idx_d[b]], rows[b], sem_g[b])
        for b in range(NBUF):
            @pl.when(live[b])
            def _(b=b):
                pltpu.make_async_copy(x_hbm.at[idx_d[b]], rows[b], sem_g[b]).wait()
                pltpu.make_async_copy(
                    negx_hbm.at[idx_s[b]], rows[b], sem_a[b]).start(add=True)
        for b in range(NBUF):
            @pl.when(live[b])
            def _(b=b):
                pltpu.make_async_copy(negx_hbm.at[idx_s[b]], rows[b], sem_a[b]).wait()
                pltpu.async_copy(rows[b], out_hbm.at[pl.ds(bases[b], C)], sem_o[b])
        return carry

    lax.fori_loop(0, NSLOT // NBUF, group_body, 0, unroll=False)
    for b in range(NBUF):
        last_j = ((trips - 1 - b) // NBUF) * NBUF + b
        pltpu.make_async_copy(
            rows[b], out_hbm.at[pl.ds((chunk0 + last_j) * C, C)],
            sem_o[b]).wait()


def kernel(x, edge_index):
    negx = _negate(x)
    src = edge_index[0]
    dst = edge_index[1]
    return _edge_grad(x, negx, src, dst)
